# baseline (speedup 1.0000x reference)
import functools

import jax
import jax.numpy as jnp
from jax import lax
from jax.experimental import pallas as pl
from jax.experimental.pallas import tpu as pltpu
from jax.experimental.pallas import tpu_sc as plsc

N_NODES = 10000
NP = 10112
NSUB = 16
RPT = NP // NSUB
K = 128
T_ES = 80
T_FS = 160
EPAD = 2 * NSUB * T_ES * K


def _make_agg_es(df):
  mesh = plsc.VectorSubcoreMesh(core_axis_name="c", subcore_axis_name="s")

  @functools.partial(
      pl.kernel,
      out_type=jax.ShapeDtypeStruct((2 * NP, df), jnp.float32),
      mesh=mesh,
      compiler_params=pltpu.CompilerParams(use_tc_tiling_on_sc=False),
      scratch_types=[
          pltpu.VMEM((T_ES // 2, K), jnp.int32),
          pltpu.VMEM((T_ES // 2, K), jnp.int32),
          pltpu.VMEM((2, K, df), jnp.float32),
          pltpu.VMEM_SHARED((NP, df), jnp.float32),
          pltpu.SemaphoreType.DMA((2,)),
      ],
  )
  def agg(tab, zer, src64, dst64, out, src_v, dst_v, rows, acc, sems):
    T = T_ES // 2
    c = lax.axis_index("c")
    s = lax.axis_index("s")
    w = c * NSUB + s
    r0 = s * RPT

    @pl.when(c == 0)
    def _():
      pltpu.sync_copy(tab.at[pl.ds(r0, RPT)], acc.at[pl.ds(r0, RPT)])

    @pl.when(c == 1)
    def _():
      pltpu.sync_copy(zer.at[pl.ds(r0, RPT)], acc.at[pl.ds(r0, RPT)])

    plsc.subcore_barrier()

    def start(j, b):
      pltpu.async_copy(tab.at[src_v.at[j]], rows.at[b], sems.at[b])

    def wait(j, b):
      pltpu.make_async_copy(tab.at[src_v.at[j]], rows.at[b],
                            sems.at[b]).wait()

    def scat(j, b):
      pltpu.sync_copy(rows.at[b], acc.at[dst_v.at[j]], add=True)

    def body(i, carry):
      j = 2 * i
      start(j + 1, 1)
      wait(j, 0)
      scat(j, 0)

      @pl.when(j + 2 < T)
      def _():
        start(j + 2, 0)

      wait(j + 1, 1)
      scat(j + 1, 1)
      return carry

    for h in range(2):
      pltpu.sync_copy(src64.at[2 * w + h], src_v)
      pltpu.sync_copy(dst64.at[2 * w + h], dst_v)
      start(0, 0)
      lax.fori_loop(0, T // 2, body, 0)

    plsc.subcore_barrier()
    pltpu.sync_copy(acc.at[pl.ds(r0, RPT)], out.at[pl.ds(c * NP + r0, RPT)])

  return agg


def _make_deg():
  mesh = plsc.VectorSubcoreMesh(core_axis_name="c", subcore_axis_name="s")

  @functools.partial(
      pl.kernel,
      out_type=jax.ShapeDtypeStruct((2 * NP, 16), jnp.float32),
      mesh=mesh,
      compiler_params=pltpu.CompilerParams(use_tc_tiling_on_sc=False),
      scratch_types=[
          pltpu.VMEM((T_ES // 2, K), jnp.int32),
          pltpu.VMEM((K, 16), jnp.float32),
          pltpu.VMEM_SHARED((NP, 16), jnp.float32),
      ],
  )
  def deg(dst64, ones_h, zer_h, out, dst_v, ones_v, acc):
    T = T_ES // 2
    c = lax.axis_index("c")
    s = lax.axis_index("s")
    w = c * NSUB + s
    r0 = s * RPT
    pltpu.sync_copy(zer_h.at[pl.ds(r0, RPT)], acc.at[pl.ds(r0, RPT)])
    pltpu.sync_copy(ones_h, ones_v)
    plsc.subcore_barrier()

    def body(j, carry):
      pltpu.sync_copy(ones_v, acc.at[dst_v.at[j]], add=True)
      return carry

    for h in range(2):
      pltpu.sync_copy(dst64.at[2 * w + h], dst_v)
      lax.fori_loop(0, T, body, 0)
    plsc.subcore_barrier()
    pltpu.sync_copy(acc.at[pl.ds(r0, RPT)], out.at[pl.ds(c * NP + r0, RPT)])

  return deg


_agg128 = _make_agg_es(128)
_agg64 = _make_agg_es(64)
_deg = _make_deg()

RB = 1264


def _pre_body(d_ref, x_ref, dinv_ref, xs_ref):
  deg = d_ref[0, :, 0:1] + d_ref[1, :, 0:1] + 1.0
  dinv = lax.rsqrt(deg)
  dinv_ref[...] = dinv
  xs = x_ref[...] * dinv
  xs_ref[0] = xs
  xs_ref[1] = xs


def _mid_body(a_ref, dinv_ref, w1_ref, b1_ref, w2_ref, gs_ref):
  dv = dinv_ref[...]
  y = (a_ref[0] + a_ref[1]) * dv
  h = jnp.dot(y, w1_ref[...], preferred_element_type=jnp.float32) + b1_ref[...]
  h = jnp.maximum(h, 0.0)
  g = jnp.dot(h, w2_ref[...], preferred_element_type=jnp.float32) * dv
  gs_ref[0] = g
  gs_ref[1] = g


def _fin_body(a_ref, dinv_ref, b2_ref, o_ref):
  l = (a_ref[0] + a_ref[1]) * dinv_ref[...] + b2_ref[...]
  m = jnp.max(l, axis=-1, keepdims=True)
  lse = m + jnp.log(jnp.sum(jnp.exp(l - m), axis=-1, keepdims=True))
  o_ref[...] = l - lse


def kernel(x, edge_index, W1, b1, W2, b2):
  n = x.shape[0]
  e = edge_index.shape[1]
  src = edge_index[0]
  dst = edge_index[1]
  pad = EPAD - e
  src_p = jnp.concatenate([src, jnp.zeros((pad,), jnp.int32)])
  pad_dst = n + (jnp.arange(pad, dtype=jnp.int32) % (NP - n))
  dst_p = jnp.concatenate([dst, pad_dst])
  src64 = src_p.reshape(4 * NSUB, T_ES // 2, K)
  dst64 = dst_p.reshape(4 * NSUB, T_ES // 2, K)
  src64 = jnp.concatenate([src64[:2 * NSUB], src64[2 * NSUB:] + NP], axis=0)

  ones_h = jnp.zeros((K, 16), jnp.float32).at[:, 0].set(1.0)
  zer16 = jnp.zeros((NP, 16), jnp.float32)
  zer128 = jnp.zeros((NP, 128), jnp.float32)
  deg16 = _deg(dst64, ones_h, zer16).reshape(2, NP, 16)

  x_p = jnp.pad(x, ((0, NP - n), (0, 0)))
  nblk = NP // RB
  dinv, xs = pl.pallas_call(
      _pre_body,
      grid=(nblk,),
      in_specs=[
          pl.BlockSpec((2, RB, 16), lambda i: (0, i, 0)),
          pl.BlockSpec((RB, 128), lambda i: (i, 0)),
      ],
      out_specs=[
          pl.BlockSpec((RB, 1), lambda i: (i, 0)),
          pl.BlockSpec((2, RB, 128), lambda i: (0, i, 0)),
      ],
      out_shape=[
          jax.ShapeDtypeStruct((NP, 1), jnp.float32),
          jax.ShapeDtypeStruct((2, NP, 128), jnp.float32),
      ],
  )(deg16, x_p)

  agg1 = _agg128(xs.reshape(2 * NP, 128), zer128, src64, dst64).reshape(2, NP, 128)

  gs = pl.pallas_call(
      _mid_body,
      grid=(nblk,),
      in_specs=[
          pl.BlockSpec((2, RB, 128), lambda i: (0, i, 0)),
          pl.BlockSpec((RB, 1), lambda i: (i, 0)),
          pl.BlockSpec((128, 128), lambda i: (0, 0)),
          pl.BlockSpec((1, 128), lambda i: (0, 0)),
          pl.BlockSpec((128, 64), lambda i: (0, 0)),
      ],
      out_specs=pl.BlockSpec((2, RB, 64), lambda i: (0, i, 0)),
      out_shape=jax.ShapeDtypeStruct((2, NP, 64), jnp.float32),
  )(agg1, dinv, W1, b1.reshape(1, -1), W2)

  zer64 = jnp.zeros((NP, 64), jnp.float32)
  agg2 = _agg64(gs.reshape(2 * NP, 64), zer64, src64, dst64).reshape(2, NP, 64)

  out = pl.pallas_call(
      _fin_body,
      grid=(nblk,),
      in_specs=[
          pl.BlockSpec((2, RB, 64), lambda i: (0, i, 0)),
          pl.BlockSpec((RB, 1), lambda i: (i, 0)),
          pl.BlockSpec((1, 64), lambda i: (0, 0)),
      ],
      out_specs=pl.BlockSpec((RB, 64), lambda i: (i, 0)),
      out_shape=jax.ShapeDtypeStruct((n, W2.shape[1]), jnp.float32),
  )(agg2, dinv, b2.reshape(1, -1))
  return out


# speedup vs baseline: 1.2689x; 1.2689x over previous
import functools

import jax
import jax.numpy as jnp
from jax import lax
from jax.experimental import pallas as pl
from jax.experimental.pallas import tpu as pltpu
from jax.experimental.pallas import tpu_sc as plsc

N_NODES = 10000
NP = 10112
NSUB = 16
RPT = NP // NSUB
K = 128
T = 160
EPAD = NSUB * T * K


def _make_agg_fs(df):
  mesh = plsc.VectorSubcoreMesh(core_axis_name="c", subcore_axis_name="s")

  @functools.partial(
      pl.kernel,
      out_type=jax.ShapeDtypeStruct((2 * NP, df), jnp.float32),
      mesh=mesh,
      compiler_params=pltpu.CompilerParams(use_tc_tiling_on_sc=False),
      scratch_types=[
          pltpu.VMEM((T, K), jnp.int32),
          pltpu.VMEM((T, K), jnp.int32),
          pltpu.VMEM((2, K, df), jnp.float32),
          pltpu.VMEM_SHARED((NP, df), jnp.float32),
          pltpu.SemaphoreType.DMA((2,)),
      ],
  )
  def agg(tab, srcfs, dst16, out, src_v, dst_v, rows, acc, sems):
    c = lax.axis_index("c")
    s = lax.axis_index("s")
    w = c * NSUB + s
    r0 = s * RPT
    pltpu.sync_copy(tab.at[pl.ds(c * NP + r0, RPT)], acc.at[pl.ds(r0, RPT)])
    pltpu.sync_copy(srcfs.at[w], src_v)
    pltpu.sync_copy(dst16.at[s], dst_v)
    plsc.subcore_barrier()

    def start(j, b):
      pltpu.async_copy(tab.at[src_v.at[j]], rows.at[b], sems.at[b])

    def wait(j, b):
      pltpu.make_async_copy(tab.at[src_v.at[j]], rows.at[b],
                            sems.at[b]).wait()

    def scat(j, b):
      pltpu.sync_copy(rows.at[b], acc.at[dst_v.at[j]], add=True)

    start(0, 0)

    def body(i, carry):
      j = 2 * i
      start(j + 1, 1)
      wait(j, 0)
      scat(j, 0)

      @pl.when(j + 2 < T)
      def _():
        start(j + 2, 0)

      wait(j + 1, 1)
      scat(j + 1, 1)
      return carry

    lax.fori_loop(0, T // 2, body, 0)
    plsc.subcore_barrier()
    pltpu.sync_copy(acc.at[pl.ds(r0, RPT)], out.at[pl.ds(c * NP + r0, RPT)])

  return agg


def _make_deg():
  mesh = plsc.VectorSubcoreMesh(core_axis_name="c", subcore_axis_name="s")

  @functools.partial(
      pl.kernel,
      out_type=jax.ShapeDtypeStruct((2 * NP, 16), jnp.float32),
      mesh=mesh,
      compiler_params=pltpu.CompilerParams(use_tc_tiling_on_sc=False),
      scratch_types=[
          pltpu.VMEM((T // 2, K), jnp.int32),
          pltpu.VMEM((K, 16), jnp.float32),
          pltpu.VMEM_SHARED((NP, 16), jnp.float32),
      ],
  )
  def deg(dst32, ones_h, zer_h, out, dst_v, ones_v, acc):
    c = lax.axis_index("c")
    s = lax.axis_index("s")
    w = c * NSUB + s
    r0 = s * RPT
    pltpu.sync_copy(zer_h.at[pl.ds(r0, RPT)], acc.at[pl.ds(r0, RPT)])
    pltpu.sync_copy(dst32.at[w], dst_v)
    pltpu.sync_copy(ones_h, ones_v)
    plsc.subcore_barrier()

    def body(j, carry):
      pltpu.sync_copy(ones_v, acc.at[dst_v.at[j]], add=True)
      return carry

    lax.fori_loop(0, T // 2, body, 0)
    plsc.subcore_barrier()
    pltpu.sync_copy(acc.at[pl.ds(r0, RPT)], out.at[pl.ds(c * NP + r0, RPT)])

  return deg


_agg64 = _make_agg_fs(64)
_agg32 = _make_agg_fs(32)
_deg = _make_deg()

RB = 1264


def _pre_body(d_ref, x_ref, dinv_ref, xs_ref):
  deg = d_ref[0, :, 0:1] + d_ref[1, :, 0:1] + 1.0
  dinv = lax.rsqrt(deg)
  dinv_ref[...] = dinv
  xs_ref[0] = x_ref[:, :64] * dinv
  xs_ref[1] = x_ref[:, 64:] * dinv


def _mid_body(a_ref, dinv_ref, w1_ref, b1_ref, w2_ref, gs_ref):
  dv = dinv_ref[...]
  y0 = a_ref[0] * dv
  y1 = a_ref[1] * dv
  h = (jnp.dot(y0, w1_ref[:64, :], preferred_element_type=jnp.float32)
       + jnp.dot(y1, w1_ref[64:, :], preferred_element_type=jnp.float32)
       + b1_ref[...])
  h = jnp.maximum(h, 0.0)
  g = jnp.dot(h, w2_ref[...], preferred_element_type=jnp.float32) * dv
  gs_ref[0] = g[:, :32]
  gs_ref[1] = g[:, 32:]


def _fin_body(a_ref, dinv_ref, b2_ref, o_ref):
  dv = dinv_ref[...]
  l0 = a_ref[0] * dv + b2_ref[:, :32]
  l1 = a_ref[1] * dv + b2_ref[:, 32:]
  m = jnp.maximum(jnp.max(l0, axis=-1, keepdims=True),
                  jnp.max(l1, axis=-1, keepdims=True))
  lse = m + jnp.log(jnp.sum(jnp.exp(l0 - m), axis=-1, keepdims=True)
                    + jnp.sum(jnp.exp(l1 - m), axis=-1, keepdims=True))
  o_ref[:, :32] = l0 - lse
  o_ref[:, 32:] = l1 - lse


def kernel(x, edge_index, W1, b1, W2, b2):
  n = x.shape[0]
  e = edge_index.shape[1]
  src = edge_index[0]
  dst = edge_index[1]
  pad = EPAD - e
  src_p = jnp.concatenate([src, jnp.zeros((pad,), jnp.int32)])
  pad_dst = n + (jnp.arange(pad, dtype=jnp.int32) % (NP - n))
  dst_p = jnp.concatenate([dst, pad_dst])
  src16 = src_p.reshape(NSUB, T, K)
  dst16 = dst_p.reshape(NSUB, T, K)
  srcfs = jnp.concatenate([src16, src16 + NP], axis=0)
  dst32 = dst_p.reshape(2 * NSUB, T // 2, K)

  ones_h = jnp.zeros((K, 16), jnp.float32).at[:, 0].set(1.0)
  zer16 = jnp.zeros((NP, 16), jnp.float32)
  deg16 = _deg(dst32, ones_h, zer16).reshape(2, NP, 16)

  x_p = jnp.pad(x, ((0, NP - n), (0, 0)))
  nblk = NP // RB
  dinv, xs = pl.pallas_call(
      _pre_body,
      grid=(nblk,),
      in_specs=[
          pl.BlockSpec((2, RB, 16), lambda i: (0, i, 0)),
          pl.BlockSpec((RB, 128), lambda i: (i, 0)),
      ],
      out_specs=[
          pl.BlockSpec((RB, 1), lambda i: (i, 0)),
          pl.BlockSpec((2, RB, 64), lambda i: (0, i, 0)),
      ],
      out_shape=[
          jax.ShapeDtypeStruct((NP, 1), jnp.float32),
          jax.ShapeDtypeStruct((2, NP, 64), jnp.float32),
      ],
  )(deg16, x_p)

  agg1 = _agg64(xs.reshape(2 * NP, 64), srcfs, dst16).reshape(2, NP, 64)

  gs = pl.pallas_call(
      _mid_body,
      grid=(nblk,),
      in_specs=[
          pl.BlockSpec((2, RB, 64), lambda i: (0, i, 0)),
          pl.BlockSpec((RB, 1), lambda i: (i, 0)),
          pl.BlockSpec((128, 128), lambda i: (0, 0)),
          pl.BlockSpec((1, 128), lambda i: (0, 0)),
          pl.BlockSpec((128, 64), lambda i: (0, 0)),
      ],
      out_specs=pl.BlockSpec((2, RB, 32), lambda i: (0, i, 0)),
      out_shape=jax.ShapeDtypeStruct((2, NP, 32), jnp.float32),
  )(agg1, dinv, W1, b1.reshape(1, -1), W2)

  agg2 = _agg32(gs.reshape(2 * NP, 32), srcfs, dst16).reshape(2, NP, 32)

  out = pl.pallas_call(
      _fin_body,
      grid=(nblk,),
      in_specs=[
          pl.BlockSpec((2, RB, 32), lambda i: (0, i, 0)),
          pl.BlockSpec((RB, 1), lambda i: (i, 0)),
          pl.BlockSpec((1, 64), lambda i: (0, 0)),
      ],
      out_specs=pl.BlockSpec((RB, 64), lambda i: (i, 0)),
      out_shape=jax.ShapeDtypeStruct((n, W2.shape[1]), jnp.float32),
  )(agg2, dinv, b2.reshape(1, -1))
  return out


# speedup vs baseline: 1.4001x; 1.1034x over previous
import functools

import jax
import jax.numpy as jnp
from jax import lax
from jax.experimental import pallas as pl
from jax.experimental.pallas import tpu as pltpu
from jax.experimental.pallas import tpu_sc as plsc

N_NODES = 10000
NP = 10240
NSUB = 16
RPT = NP // NSUB
K = 128
T = 160
EPAD = NSUB * T * K


def _make_agg_fs(df):
  mesh = plsc.VectorSubcoreMesh(core_axis_name="c", subcore_axis_name="s")

  @functools.partial(
      pl.kernel,
      out_type=jax.ShapeDtypeStruct((2 * NP, df), jnp.float32),
      mesh=mesh,
      compiler_params=pltpu.CompilerParams(use_tc_tiling_on_sc=False),
      scratch_types=[
          pltpu.VMEM((T, K), jnp.int32),
          pltpu.VMEM((T, K), jnp.int32),
          pltpu.VMEM((2, K, df), jnp.float32),
          pltpu.VMEM_SHARED((NP, df), jnp.float32),
          pltpu.SemaphoreType.DMA((2,)),
      ],
  )
  def agg(tab, srcfs, dst16, out, src_v, dst_v, rows, acc, sems):
    c = lax.axis_index("c")
    s = lax.axis_index("s")
    w = c * NSUB + s
    r0 = s * RPT
    pltpu.sync_copy(tab.at[pl.ds(c * NP + r0, RPT)], acc.at[pl.ds(r0, RPT)])
    pltpu.sync_copy(srcfs.at[w], src_v)
    pltpu.sync_copy(dst16.at[s], dst_v)
    plsc.subcore_barrier()

    def start(j, b):
      pltpu.async_copy(tab.at[src_v.at[j]], rows.at[b], sems.at[b])

    def wait(j, b):
      pltpu.make_async_copy(tab.at[src_v.at[j]], rows.at[b],
                            sems.at[b]).wait()

    def scat(j, b):
      pltpu.sync_copy(rows.at[b], acc.at[dst_v.at[j]], add=True)

    start(0, 0)

    def body(i, carry):
      j = 2 * i
      start(j + 1, 1)
      wait(j, 0)
      scat(j, 0)

      @pl.when(j + 2 < T)
      def _():
        start(j + 2, 0)

      wait(j + 1, 1)
      scat(j + 1, 1)
      return carry

    lax.fori_loop(0, T // 2, body, 0)
    plsc.subcore_barrier()
    pltpu.sync_copy(acc.at[pl.ds(r0, RPT)], out.at[pl.ds(c * NP + r0, RPT)])

  return agg


def _make_deg():
  mesh = plsc.VectorSubcoreMesh(core_axis_name="c", subcore_axis_name="s")

  @functools.partial(
      pl.kernel,
      out_type=jax.ShapeDtypeStruct((2 * NP, 16), jnp.float32),
      mesh=mesh,
      compiler_params=pltpu.CompilerParams(use_tc_tiling_on_sc=False),
      scratch_types=[
          pltpu.VMEM((T // 2, K), jnp.int32),
          pltpu.VMEM((K, 16), jnp.float32),
          pltpu.VMEM_SHARED((NP, 16), jnp.float32),
      ],
  )
  def deg(dst32, ones_h, zer_h, out, dst_v, ones_v, acc):
    c = lax.axis_index("c")
    s = lax.axis_index("s")
    w = c * NSUB + s
    r0 = s * RPT
    pltpu.sync_copy(zer_h.at[pl.ds(r0, RPT)], acc.at[pl.ds(r0, RPT)])
    pltpu.sync_copy(dst32.at[w], dst_v)
    pltpu.sync_copy(ones_h, ones_v)
    plsc.subcore_barrier()

    def body(j, carry):
      pltpu.sync_copy(ones_v, acc.at[dst_v.at[j]], add=True)
      return carry

    lax.fori_loop(0, T // 2, body, 0)
    plsc.subcore_barrier()
    pltpu.sync_copy(acc.at[pl.ds(r0, RPT)], out.at[pl.ds(c * NP + r0, RPT)])

  return deg


_agg64 = _make_agg_fs(64)
_agg32 = _make_agg_fs(32)
_deg = _make_deg()

RB = 1280


def _pre_body(d_ref, x_ref, dinv_ref, xs_ref):
  deg = d_ref[0, :, 0:1] + d_ref[1, :, 0:1] + 1.0
  dinv = lax.rsqrt(deg)
  dinv_ref[...] = dinv
  xs_ref[0] = x_ref[:, :64] * dinv
  xs_ref[1] = x_ref[:, 64:] * dinv


def _mid_body(a_ref, dinv_ref, w1_ref, b1_ref, w2_ref, gs_ref):
  dv = dinv_ref[...]
  y0 = a_ref[0] * dv
  y1 = a_ref[1] * dv
  h = (jnp.dot(y0, w1_ref[:64, :], preferred_element_type=jnp.float32)
       + jnp.dot(y1, w1_ref[64:, :], preferred_element_type=jnp.float32)
       + b1_ref[...])
  h = jnp.maximum(h, 0.0)
  g = jnp.dot(h, w2_ref[...], preferred_element_type=jnp.float32) * dv
  row = (jax.lax.broadcasted_iota(jnp.int32, (RB, 1), 0)
         + pl.program_id(0) * RB)
  g = jnp.where(row < N_NODES, g, 0.0)
  gs_ref[0] = g[:, :32]
  gs_ref[1] = g[:, 32:]


def _fin_body(a_ref, dinv_ref, b2_ref, o_ref):
  dv = dinv_ref[...]
  l0 = a_ref[0] * dv + b2_ref[:, :32]
  l1 = a_ref[1] * dv + b2_ref[:, 32:]
  m = jnp.maximum(jnp.max(l0, axis=-1, keepdims=True),
                  jnp.max(l1, axis=-1, keepdims=True))
  lse = m + jnp.log(jnp.sum(jnp.exp(l0 - m), axis=-1, keepdims=True)
                    + jnp.sum(jnp.exp(l1 - m), axis=-1, keepdims=True))
  o_ref[:, :32] = l0 - lse
  o_ref[:, 32:] = l1 - lse


def kernel(x, edge_index, W1, b1, W2, b2):
  n = x.shape[0]
  e = edge_index.shape[1]
  src = edge_index[0]
  dst = edge_index[1]
  pad = EPAD - e
  src_p = jnp.concatenate([src, jnp.full((pad,), n, jnp.int32)])
  dst_p = jnp.concatenate([dst, jnp.arange(pad, dtype=jnp.int32) % n])
  dstd_p = jnp.concatenate([dst, n + jnp.arange(pad, dtype=jnp.int32)
                            % (NP - n)])
  src16 = src_p.reshape(NSUB, T, K)
  dst16 = dst_p.reshape(NSUB, T, K)
  srcfs = jnp.concatenate([src16, src16 + NP], axis=0)
  dst32 = dstd_p.reshape(2 * NSUB, T // 2, K)

  ones_h = jnp.zeros((K, 16), jnp.float32).at[:, 0].set(1.0)
  zer16 = jnp.zeros((NP, 16), jnp.float32)
  deg16 = _deg(dst32, ones_h, zer16).reshape(2, NP, 16)

  x_p = jnp.pad(x, ((0, NP - n), (0, 0)))
  nblk = NP // RB
  dinv, xs = pl.pallas_call(
      _pre_body,
      grid=(nblk,),
      in_specs=[
          pl.BlockSpec((2, RB, 16), lambda i: (0, i, 0)),
          pl.BlockSpec((RB, 128), lambda i: (i, 0)),
      ],
      out_specs=[
          pl.BlockSpec((RB, 1), lambda i: (i, 0)),
          pl.BlockSpec((2, RB, 64), lambda i: (0, i, 0)),
      ],
      out_shape=[
          jax.ShapeDtypeStruct((NP, 1), jnp.float32),
          jax.ShapeDtypeStruct((2, NP, 64), jnp.float32),
      ],
  )(deg16, x_p)

  agg1 = _agg64(xs.reshape(2 * NP, 64), srcfs, dst16).reshape(2, NP, 64)

  gs = pl.pallas_call(
      _mid_body,
      grid=(nblk,),
      in_specs=[
          pl.BlockSpec((2, RB, 64), lambda i: (0, i, 0)),
          pl.BlockSpec((RB, 1), lambda i: (i, 0)),
          pl.BlockSpec((128, 128), lambda i: (0, 0)),
          pl.BlockSpec((1, 128), lambda i: (0, 0)),
          pl.BlockSpec((128, 64), lambda i: (0, 0)),
      ],
      out_specs=pl.BlockSpec((2, RB, 32), lambda i: (0, i, 0)),
      out_shape=jax.ShapeDtypeStruct((2, NP, 32), jnp.float32),
  )(agg1, dinv, W1, b1.reshape(1, -1), W2)

  agg2 = _agg32(gs.reshape(2 * NP, 32), srcfs, dst16).reshape(2, NP, 32)

  out = pl.pallas_call(
      _fin_body,
      grid=(nblk,),
      in_specs=[
          pl.BlockSpec((2, RB, 32), lambda i: (0, i, 0)),
          pl.BlockSpec((RB, 1), lambda i: (i, 0)),
          pl.BlockSpec((1, 64), lambda i: (0, 0)),
      ],
      out_specs=pl.BlockSpec((RB, 64), lambda i: (i, 0)),
      out_shape=jax.ShapeDtypeStruct((n, W2.shape[1]), jnp.float32),
  )(agg2, dinv, b2.reshape(1, -1))
  return out


# speedup vs baseline: 1.6694x; 1.1923x over previous
import functools

import jax
import jax.numpy as jnp
from jax import lax
from jax.experimental import pallas as pl
from jax.experimental.pallas import tpu as pltpu
from jax.experimental.pallas import tpu_sc as plsc

N_NODES = 10000
NP = 10240
NSUB = 16
RPT = NP // NSUB
K = 128
T = 158
EPAD = NSUB * T * K


def _make_agg_fs(df):
  mesh = plsc.VectorSubcoreMesh(core_axis_name="c", subcore_axis_name="s")

  @functools.partial(
      pl.kernel,
      out_type=jax.ShapeDtypeStruct((2 * NP, df), jnp.float32),
      mesh=mesh,
      compiler_params=pltpu.CompilerParams(use_tc_tiling_on_sc=False),
      scratch_types=[
          pltpu.VMEM((T, K), jnp.int32),
          pltpu.VMEM((T, K), jnp.int32),
          pltpu.VMEM((2, K, df), jnp.float32),
          pltpu.VMEM_SHARED((NP, df), jnp.float32),
          pltpu.SemaphoreType.DMA((2,)),
      ],
  )
  def agg(tab, srcfs, dst16, out, src_v, dst_v, rows, acc, sems):
    c = lax.axis_index("c")
    s = lax.axis_index("s")
    w = c * NSUB + s
    r0 = s * RPT
    pltpu.sync_copy(tab.at[pl.ds(c * NP + r0, RPT)], acc.at[pl.ds(r0, RPT)])
    pltpu.sync_copy(srcfs.at[w], src_v)
    pltpu.sync_copy(dst16.at[s], dst_v)
    plsc.subcore_barrier()

    def start(j, b):
      pltpu.async_copy(tab.at[src_v.at[j]], rows.at[b], sems.at[b])

    def wait(j, b):
      pltpu.make_async_copy(tab.at[src_v.at[j]], rows.at[b],
                            sems.at[b]).wait()

    def scat(j, b):
      pltpu.sync_copy(rows.at[b], acc.at[dst_v.at[j]], add=True)

    start(0, 0)

    def body(i, carry):
      j = 2 * i
      start(j + 1, 1)
      wait(j, 0)
      scat(j, 0)

      @pl.when(j + 2 < T)
      def _():
        start(j + 2, 0)

      wait(j + 1, 1)
      scat(j + 1, 1)
      return carry

    lax.fori_loop(0, T // 2, body, 0)
    plsc.subcore_barrier()
    pltpu.sync_copy(acc.at[pl.ds(r0, RPT)], out.at[pl.ds(c * NP + r0, RPT)])

  return agg


def _make_deg():
  mesh = plsc.VectorSubcoreMesh(core_axis_name="c", subcore_axis_name="s")

  @functools.partial(
      pl.kernel,
      out_type=jax.ShapeDtypeStruct((2 * NP, 16), jnp.float32),
      mesh=mesh,
      compiler_params=pltpu.CompilerParams(use_tc_tiling_on_sc=False),
      scratch_types=[
          pltpu.VMEM((T // 2, K), jnp.int32),
          pltpu.VMEM((K, 16), jnp.float32),
          pltpu.VMEM_SHARED((NP, 16), jnp.float32),
      ],
  )
  def deg(dst32, ones_h, zer_h, out, dst_v, ones_v, acc):
    c = lax.axis_index("c")
    s = lax.axis_index("s")
    w = c * NSUB + s
    r0 = s * RPT
    pltpu.sync_copy(zer_h.at[pl.ds(r0, RPT)], acc.at[pl.ds(r0, RPT)])
    pltpu.sync_copy(dst32.at[w], dst_v)
    pltpu.sync_copy(ones_h, ones_v)
    plsc.subcore_barrier()

    def body(j, carry):
      pltpu.sync_copy(ones_v, acc.at[dst_v.at[j]], add=True)
      return carry

    lax.fori_loop(0, T // 2, body, 0)
    plsc.subcore_barrier()
    pltpu.sync_copy(acc.at[pl.ds(r0, RPT)], out.at[pl.ds(c * NP + r0, RPT)])

  return deg


_agg64 = _make_agg_fs(64)
_agg32 = _make_agg_fs(32)
_deg = _make_deg()

RB = 1280


def _pre_body(d_ref, x_ref, dinv_ref, xs_ref):
  deg = d_ref[0, :, 0:1] + d_ref[1, :, 0:1] + 1.0
  dinv = lax.rsqrt(deg)
  dinv_ref[...] = dinv
  xs_ref[0] = x_ref[:, :64] * dinv
  xs_ref[1] = x_ref[:, 64:] * dinv


def _mid_body(a_ref, dinv_ref, w1_ref, b1_ref, w2_ref, gs_ref):
  dv = dinv_ref[...]
  y0 = a_ref[0] * dv
  y1 = a_ref[1] * dv
  h = (jnp.dot(y0, w1_ref[:64, :], preferred_element_type=jnp.float32)
       + jnp.dot(y1, w1_ref[64:, :], preferred_element_type=jnp.float32)
       + b1_ref[...])
  h = jnp.maximum(h, 0.0)
  g = jnp.dot(h, w2_ref[...], preferred_element_type=jnp.float32) * dv
  row = (jax.lax.broadcasted_iota(jnp.int32, (RB, 1), 0)
         + pl.program_id(0) * RB)
  g = jnp.where(row < N_NODES, g, 0.0)
  gs_ref[0] = g[:, :32]
  gs_ref[1] = g[:, 32:]


def _fin_body(a_ref, dinv_ref, b2_ref, o_ref):
  dv = dinv_ref[...]
  l0 = a_ref[0] * dv + b2_ref[:, :32]
  l1 = a_ref[1] * dv + b2_ref[:, 32:]
  m = jnp.maximum(jnp.max(l0, axis=-1, keepdims=True),
                  jnp.max(l1, axis=-1, keepdims=True))
  lse = m + jnp.log(jnp.sum(jnp.exp(l0 - m), axis=-1, keepdims=True)
                    + jnp.sum(jnp.exp(l1 - m), axis=-1, keepdims=True))
  o_ref[:, :32] = l0 - lse
  o_ref[:, 32:] = l1 - lse


def kernel(x, edge_index, W1, b1, W2, b2):
  n = x.shape[0]
  e = edge_index.shape[1]
  src = edge_index[0]
  dst = edge_index[1]
  pad = EPAD - e
  src_p = jnp.concatenate([src, jnp.full((pad,), n, jnp.int32)])
  dst_p = jnp.concatenate([dst, jnp.arange(pad, dtype=jnp.int32) % n])
  dstd_p = jnp.concatenate([dst, n + jnp.arange(pad, dtype=jnp.int32)
                            % (NP - n)])
  src16 = src_p.reshape(NSUB, T, K)
  dst16 = dst_p.reshape(NSUB, T, K)
  srcfs = jnp.concatenate([src16, src16 + NP], axis=0)
  dst32 = dstd_p.reshape(2 * NSUB, T // 2, K)

  ones_h = jnp.zeros((K, 16), jnp.float32).at[:, 0].set(1.0)
  zer16 = jnp.zeros((NP, 16), jnp.float32)
  deg16 = _deg(dst32, ones_h, zer16).reshape(2, NP, 16)

  x_p = jnp.pad(x, ((0, NP - n), (0, 0)))
  nblk = NP // RB
  dinv, xs = pl.pallas_call(
      _pre_body,
      grid=(nblk,),
      in_specs=[
          pl.BlockSpec((2, RB, 16), lambda i: (0, i, 0)),
          pl.BlockSpec((RB, 128), lambda i: (i, 0)),
      ],
      out_specs=[
          pl.BlockSpec((RB, 1), lambda i: (i, 0)),
          pl.BlockSpec((2, RB, 64), lambda i: (0, i, 0)),
      ],
      out_shape=[
          jax.ShapeDtypeStruct((NP, 1), jnp.float32),
          jax.ShapeDtypeStruct((2, NP, 64), jnp.float32),
      ],
  )(deg16, x_p)

  agg1 = _agg64(xs.reshape(2 * NP, 64), srcfs, dst16).reshape(2, NP, 64)

  gs = pl.pallas_call(
      _mid_body,
      grid=(nblk,),
      in_specs=[
          pl.BlockSpec((2, RB, 64), lambda i: (0, i, 0)),
          pl.BlockSpec((RB, 1), lambda i: (i, 0)),
          pl.BlockSpec((128, 128), lambda i: (0, 0)),
          pl.BlockSpec((1, 128), lambda i: (0, 0)),
          pl.BlockSpec((128, 64), lambda i: (0, 0)),
      ],
      out_specs=pl.BlockSpec((2, RB, 32), lambda i: (0, i, 0)),
      out_shape=jax.ShapeDtypeStruct((2, NP, 32), jnp.float32),
  )(agg1, dinv, W1, b1.reshape(1, -1), W2)

  agg2 = _agg32(gs.reshape(2 * NP, 32), srcfs, dst16).reshape(2, NP, 32)

  out = pl.pallas_call(
      _fin_body,
      grid=(nblk,),
      in_specs=[
          pl.BlockSpec((2, RB, 32), lambda i: (0, i, 0)),
          pl.BlockSpec((RB, 1), lambda i: (i, 0)),
          pl.BlockSpec((1, 64), lambda i: (0, 0)),
      ],
      out_specs=pl.BlockSpec((RB, 64), lambda i: (i, 0)),
      out_shape=jax.ShapeDtypeStruct((n, W2.shape[1]), jnp.float32),
  )(agg2, dinv, b2.reshape(1, -1))
  return out
